# baseline (speedup 1.0000x reference)
import functools

import jax
import jax.numpy as jnp
from jax import lax
from jax.experimental import pallas as pl
from jax.experimental.pallas import tpu as pltpu
from jax.experimental.pallas import tpu_sc as plsc

_N = 10000
_NP = 10112
_HW = 65536
_E = 160000
_C = 128
_CA = 144
_F1D = 16
_NC = 2
_NS = 16
_NW = _NC * _NS
_CH = 128


def _leaky(x):
    return jnp.where(x >= 0, x, 0.01 * x)


def _mm_call(x, w, b, act, blk):
    m, k = x.shape
    n = w.shape[1]
    assert m % blk == 0

    def body(x_ref, w_ref, b_ref, o_ref):
        y = jnp.dot(x_ref[...], w_ref[...], preferred_element_type=jnp.float32)
        y = y + b_ref[...]
        if act:
            y = _leaky(y)
        o_ref[...] = y

    return pl.pallas_call(
        body,
        grid=(m // blk,),
        in_specs=[
            pl.BlockSpec((blk, k), lambda i: (i, 0)),
            pl.BlockSpec((k, n), lambda i: (0, 0)),
            pl.BlockSpec((1, n), lambda i: (0, 0)),
        ],
        out_specs=pl.BlockSpec((blk, n), lambda i: (i, 0)),
        out_shape=jax.ShapeDtypeStruct((m, n), jnp.float32),
    )(x, w, b.reshape(1, n))


_WP = 272
_CB = 5888
_CNB = 12
_GPR = 272 + _CB * _CNB + 272


def _conv2_call(gp, w2m, b2):
    def body(gp_ref, w_ref, b_ref, o_ref):
        i = pl.program_id(0)
        acc = jnp.zeros((_CB, _C), jnp.float32) + b_ref[...]
        for dh in range(3):
            blk = gp_ref[pl.ds(i * _CB + _WP * dh, _CB), :]
            acc = acc + jnp.dot(blk, w_ref[pl.ds(96 * dh, 96), :],
                                preferred_element_type=jnp.float32)
        o_ref[...] = acc

    return pl.pallas_call(
        body,
        grid=(_CNB,),
        in_specs=[
            pl.BlockSpec((_GPR, 96), lambda i: (0, 0)),
            pl.BlockSpec((288, _C), lambda i: (0, 0)),
            pl.BlockSpec((1, _C), lambda i: (0, 0)),
        ],
        out_specs=pl.BlockSpec((_CB, _C), lambda i: (i, 0)),
        out_shape=jax.ShapeDtypeStruct((_CB * _CNB, _C), jnp.float32),
    )(gp, w2m, b2.reshape(1, _C))


def _merge_call(s0, s1, c0, c1):
    def body(a_ref, b_ref, ca_ref, cb_ref, o_ref):
        t = a_ref[...] + b_ref[...]
        cnt = jnp.maximum(ca_ref[...] + cb_ref[...], 1.0)
        o_ref[...] = t / cnt

    return pl.pallas_call(
        body,
        out_shape=jax.ShapeDtypeStruct((_NP, _C), jnp.float32),
    )(s0, s1, c0, c1)


def _node_call(a0, a1, w1, b1, w2):
    blk = 1264

    def body(a0_ref, a1_ref, w1_ref, b1_ref, w2_ref, nf_ref, g_ref):
        x = a0_ref[...] + a1_ref[...]
        nf = _leaky(jnp.dot(x, w1_ref[...], preferred_element_type=jnp.float32)
                    + b1_ref[...])
        nf_ref[...] = nf
        g_ref[...] = jnp.dot(nf, w2_ref[...], preferred_element_type=jnp.float32)

    return pl.pallas_call(
        body,
        grid=(_NP // blk,),
        in_specs=[
            pl.BlockSpec((blk, _C), lambda i: (i, 0)),
            pl.BlockSpec((blk, _C), lambda i: (i, 0)),
            pl.BlockSpec((_C, _C), lambda i: (0, 0)),
            pl.BlockSpec((1, _C), lambda i: (0, 0)),
            pl.BlockSpec((_C, _C), lambda i: (0, 0)),
        ],
        out_specs=[
            pl.BlockSpec((blk, _C), lambda i: (i, 0)),
            pl.BlockSpec((blk, _C), lambda i: (i, 0)),
        ],
        out_shape=[
            jax.ShapeDtypeStruct((_NP, _C), jnp.float32),
            jax.ShapeDtypeStruct((_NP, _C), jnp.float32),
        ],
    )(a0, a1, w1, b1.reshape(1, _C), w2)


def _final_call(s1, s2, ew2, ef1d, w_bot, ec1_b, ec2_b, vmat, v1d, bvec,
                u0, u1, u2, ab, kq):
    blk = 3200
    rb = blk // 128
    sigma = 0.2
    sq2 = 1.4142135623730951
    inf = float('inf')

    def body(s1_ref, s2_ref, ew_ref, e1d_ref, wb_ref, b1_ref, b2_ref,
             v_ref, v1_ref, bv_ref, u0_ref, u1_ref, u2_ref, ab_ref, kq_ref,
             p_ref, q_ref, vv_ref, sa_ref, qp_ref):
        ew = ew_ref[...]
        ef1 = _leaky(ew * s1_ref[...] + b1_ref[...])
        z = _leaky(ew * s2_ref[...] +
                   jnp.dot(ef1.astype(jnp.bfloat16),
                           wb_ref[...].astype(jnp.bfloat16),
                           preferred_element_type=jnp.float32)
                   + b2_ref[...])
        o = (jnp.dot(z, v_ref[...], preferred_element_type=jnp.float32)
             + jnp.dot(e1d_ref[...], v1_ref[...],
                       preferred_element_type=jnp.float32)
             + bv_ref[...])
        p = jax.nn.sigmoid(jnp.reshape(o[:, 0:1], (rb, 128)))
        base = jnp.reshape(o[:, 1:2], (rb, 128))
        lower = (0.0 - p) / sigma
        upper = (1.0 - p) / sigma
        ea = lax.erf(lower / sq2)
        eb = lax.erf(upper / sq2)
        lo_n = lax.nextafter(lower, inf)
        hi_n = lax.nextafter(upper, -inf)

        def fold2(u):
            return jnp.reshape(u, (rb, 128))

        def samp(u):
            uu = jnp.maximum(ea, u * (eb - ea) + ea)
            t = sq2 * lax.erf_inv(uu)
            return p + sigma * jnp.clip(t, lo_n, hi_n)

        act0 = samp(fold2(u0_ref[...]))
        act1 = samp(fold2(u1_ref[...]))
        sact = samp(fold2(u2_ref[...]))
        mean = 0.5 * (act0 + act1)
        abv = fold2(ab_ref[...])
        kqv = kq_ref[...]

        def unfold(x):
            return jnp.reshape(x, (blk, 1))

        p_ref[...] = unfold(p)
        q_ref[...] = unfold(base + kqv * (2.0 * abv - mean))
        vv_ref[...] = unfold(base + kqv * abv)
        sa_ref[...] = unfold(sact)
        qp_ref[...] = unfold(base + kqv * (2.0 * sact - mean))

    fold = pl.BlockSpec((blk, 1), lambda i: (i, 0))
    fshape = jax.ShapeDtypeStruct((_E, 1), jnp.float32)
    return pl.pallas_call(
        body,
        grid=(_E // blk,),
        in_specs=[
            pl.BlockSpec((blk, _C), lambda i: (i, 0)),
            pl.BlockSpec((blk, _C), lambda i: (i, 0)),
            pl.BlockSpec((blk, 1), lambda i: (i, 0)),
            pl.BlockSpec((blk, _F1D), lambda i: (i, 0)),
            pl.BlockSpec((_C, _C), lambda i: (0, 0)),
            pl.BlockSpec((1, _C), lambda i: (0, 0)),
            pl.BlockSpec((1, _C), lambda i: (0, 0)),
            pl.BlockSpec((_C, 2), lambda i: (0, 0)),
            pl.BlockSpec((_F1D, 2), lambda i: (0, 0)),
            pl.BlockSpec((1, 2), lambda i: (0, 0)),
            fold, fold, fold, fold,
            pl.BlockSpec((1, 1), lambda i: (0, 0)),
        ],
        out_specs=[fold, fold, fold, fold, fold],
        out_shape=[fshape, fshape, fshape, fshape, fshape],
    )(s1, s2, ew2, ef1d, w_bot, ec1_b, ec2_b, vmat, v1d, bvec,
      u0, u1, u2, ab, kq)


_MESH = dict(core_axis_name="c", subcore_axis_name="s")


def _zero_vmem(ref, nrows, ncols):
    zv = jnp.zeros((16,), jnp.float32)
    npc = ncols // 16

    def zb(i, carry):
        ref[i // npc, pl.ds((i % npc) * 16, 16)] = zv
        return carry

    lax.fori_loop(0, nrows * npc, zb, 0)


def _zero_acc(acc, zbuf, sid, rows_per_tile):
    base = sid * rows_per_tile
    nfull = rows_per_tile // _CH
    rem = rows_per_tile - nfull * _CH
    for t in range(nfull):
        pltpu.sync_copy(zbuf, acc.at[pl.ds(base + t * _CH, _CH)])
    if rem:
        pltpu.sync_copy(zbuf.at[pl.ds(0, rem)],
                        acc.at[pl.ds(base + nfull * _CH, rem)])


def _sc_pool(pix, sp_indices):
    mesh = plsc.VectorSubcoreMesh(**_MESH)
    nchunk = _HW // _CH // _NW
    rpt = _NP // _NS

    @functools.partial(
        pl.kernel,
        out_type=[
            jax.ShapeDtypeStruct((_NC, _NP, _C), jnp.float32),
            jax.ShapeDtypeStruct((_NC, _NP, _C), jnp.float32),
        ],
        mesh=mesh,
        scratch_types=[
            pltpu.VMEM((_CH, _C), jnp.float32),
            pltpu.VMEM((_CH, _C), jnp.float32),
            pltpu.VMEM((_CH,), jnp.int32),
            pltpu.VMEM_SHARED((_NP, _C), jnp.float32),
            pltpu.SemaphoreType.DMA,
        ],
    )
    def k(pix_hbm, sp_hbm, sum_hbm, cnt_hbm, zbuf, rows, idx, acc, sem):
        c = lax.axis_index("c")
        s = lax.axis_index("s")
        w = c * _NS + s
        _zero_vmem(zbuf, _CH, _C)
        _zero_acc(acc, zbuf, s, rpt)
        plsc.subcore_barrier()

        def chunk(j, carry):
            k = w * nchunk + j
            off = pl.multiple_of(k * _CH, 8)
            prow = pl.multiple_of(_WP * (k // 2) + _WP + _CH * (k % 2) + 8, 8)
            pltpu.sync_copy(sp_hbm.at[pl.ds(off, _CH)], idx)
            pltpu.sync_copy(pix_hbm.at[pl.ds(prow, _CH)], rows)
            pltpu.sync_copy(rows, acc.at[idx], add=True)
            return carry

        lax.fori_loop(0, nchunk, chunk, 0)
        plsc.subcore_barrier()
        pltpu.sync_copy(acc.at[pl.ds(s * rpt, rpt)],
                        sum_hbm.at[c, pl.ds(s * rpt, rpt)])
        plsc.subcore_barrier()
        _zero_acc(acc, zbuf, s, rpt)
        one = jnp.ones((16,), jnp.float32)

        def ob(i, carry):
            rows[i // 8, pl.ds((i % 8) * 16, 16)] = one
            return carry

        lax.fori_loop(0, _CH * 8, ob, 0)
        plsc.subcore_barrier()

        def chunk2(j, carry):
            off = pl.multiple_of((w * nchunk + j) * _CH, 8)
            pltpu.sync_copy(sp_hbm.at[pl.ds(off, _CH)], idx)
            pltpu.sync_copy(rows, acc.at[idx], add=True)
            return carry

        lax.fori_loop(0, nchunk, chunk2, 0)
        plsc.subcore_barrier()
        pltpu.sync_copy(acc.at[pl.ds(s * rpt, rpt)],
                        cnt_hbm.at[c, pl.ds(s * rpt, rpt)])

    return k(pix, sp_indices)


def _edge_off(w, j):
    return pl.multiple_of((w + _NW * j) * _CH, 8)


def _edge_nch(w):
    return jnp.where(w < 2, 40, 39)


def _edge_pipeline(w, prefetch_fn, process_fn):
    nch = _edge_nch(w)
    prefetch_fn(_edge_off(w, 0), 0)

    def body(jj, carry):
        for b in range(2):
            j = jj * 2 + b

            @pl.when(j < nch)
            def _():
                @pl.when(j + 1 < nch)
                def _():
                    prefetch_fn(_edge_off(w, j + 1), 1 - b)

                process_fn(_edge_off(w, j), b)
        return carry

    lax.fori_loop(0, 20, body, 0)


def _sc_scatter(table, src, dst, ang):
    mesh = plsc.VectorSubcoreMesh(**_MESH)
    rpt = _NP // _NS

    @functools.partial(
        pl.kernel,
        out_type=jax.ShapeDtypeStruct((_NC, _NP, _C), jnp.float32),
        mesh=mesh,
        scratch_types=[
            pltpu.VMEM((_CH, _C), jnp.float32),
            pltpu.VMEM((_CH, _C), jnp.float32),
            pltpu.VMEM((_CH, _C), jnp.float32),
            pltpu.VMEM((_CH,), jnp.int32),
            pltpu.VMEM((_CH,), jnp.int32),
            pltpu.VMEM((_CH,), jnp.int32),
            pltpu.VMEM((_CH,), jnp.int32),
            pltpu.VMEM((_CH,), jnp.float32),
            pltpu.VMEM((_CH,), jnp.float32),
            pltpu.VMEM_SHARED((_NP, _C), jnp.float32),
            pltpu.SemaphoreType.DMA,
            pltpu.SemaphoreType.DMA,
        ],
    )
    def k(tab_hbm, src_hbm, dst_hbm, ang_hbm, out_hbm,
          zbuf, rows0, rows1, si0, si1, di0, di1, av0, av1, acc,
          semg0, semg1):
        c = lax.axis_index("c")
        s = lax.axis_index("s")
        w = c * _NS + s
        rows = (rows0, rows1)
        si = (si0, si1)
        di = (di0, di1)
        av = (av0, av1)
        semg = (semg0, semg1)
        _zero_vmem(zbuf, _CH, _C)
        _zero_acc(acc, zbuf, s, rpt)
        plsc.subcore_barrier()

        dnums = lax.GatherDimensionNumbers(
            offset_dims=(), collapsed_slice_dims=(0,), start_index_map=(0,))

        def prefetch(off, b):
            pltpu.sync_copy(src_hbm.at[pl.ds(off, _CH)], si[b])
            pltpu.sync_copy(dst_hbm.at[pl.ds(off, _CH)], di[b])
            pltpu.sync_copy(ang_hbm.at[pl.ds(off, _CH)], av[b])
            pltpu.async_copy(tab_hbm.at[si[b]], rows[b], semg[b])

        def process(off, b):
            pltpu.make_async_copy(tab_hbm.at[si[b]], rows[b], semg[b]).wait()

            def rb(g, carry):
                a16 = av[b][pl.ds(g * 16, 16)]
                for l in range(16):
                    a = lax.gather(
                        a16, jnp.full((16, 1), l, jnp.int32), dnums, (1,),
                        mode=lax.GatherScatterMode.PROMISE_IN_BOUNDS)
                    r = g * 16 + l
                    for k8 in range(_C // 16):
                        sl = pl.ds(k8 * 16, 16)
                        rows[b][r, sl] = rows[b][r, sl] * a
                return carry

            lax.fori_loop(0, _CH // 16, rb, 0)
            pltpu.sync_copy(rows[b], acc.at[di[b]], add=True)

        _edge_pipeline(w, prefetch, process)
        plsc.subcore_barrier()
        pltpu.sync_copy(acc.at[pl.ds(s * rpt, rpt)],
                        out_hbm.at[c, pl.ds(s * rpt, rpt)])

    return k(table, src, dst, ang)


def _sc_pair(table, src, dst):
    mesh = plsc.VectorSubcoreMesh(**_MESH)

    @functools.partial(
        pl.kernel,
        out_type=jax.ShapeDtypeStruct((_E, _C), jnp.float32),
        mesh=mesh,
        scratch_types=[
            pltpu.VMEM((_CH, _C), jnp.float32),
            pltpu.VMEM((_CH, _C), jnp.float32),
            pltpu.VMEM((_CH, _C), jnp.float32),
            pltpu.VMEM((_CH, _C), jnp.float32),
            pltpu.VMEM((_CH,), jnp.int32),
            pltpu.VMEM((_CH,), jnp.int32),
            pltpu.VMEM((_CH,), jnp.int32),
            pltpu.VMEM((_CH,), jnp.int32),
            pltpu.SemaphoreType.DMA,
            pltpu.SemaphoreType.DMA,
        ],
    )
    def k(tab_hbm, src_hbm, dst_hbm, out_hbm,
          ra0, ra1, rb0, rb1, si0, si1, di0, di1, semg0, semg1):
        c = lax.axis_index("c")
        s = lax.axis_index("s")
        w = c * _NS + s
        ra = (ra0, ra1)
        rb = (rb0, rb1)
        si = (si0, si1)
        di = (di0, di1)
        semg = (semg0, semg1)

        def prefetch(off, b):
            pltpu.sync_copy(src_hbm.at[pl.ds(off, _CH)], si[b])
            pltpu.sync_copy(dst_hbm.at[pl.ds(off, _CH)], di[b])
            pltpu.async_copy(tab_hbm.at[si[b]], ra[b], semg[b])
            pltpu.async_copy(tab_hbm.at[di[b]], rb[b], semg[b])

        def process(off, b):
            pltpu.make_async_copy(tab_hbm.at[si[b]], ra[b], semg[b]).wait()
            pltpu.make_async_copy(tab_hbm.at[di[b]], rb[b], semg[b]).wait()

            def rr(r, carry):
                for k8 in range(_C // 16):
                    sl = pl.ds(k8 * 16, 16)
                    ra[b][r, sl] = ra[b][r, sl] + rb[b][r, sl]
                return carry

            lax.fori_loop(0, _CH, rr, 0)
            pltpu.sync_copy(ra[b], out_hbm.at[pl.ds(off, _CH)])

        _edge_pipeline(w, prefetch, process)

    return k(table, src, dst)


def kernel(edge_weights, raw1, raw2, action_behav, angles, edge_features_1d,
           conv1_w, conv1_b, conv2_w, conv2_b, nc1_w, nc1_b, ec1_w, ec1_b,
           nc2_w, nc2_b, ec2_w, ec2_b, p1_w, p1_b, p2_w, p2_b, q1_w, q1_b,
           q2_w, q2_b, sp_indices, edge_index):
    f32 = jnp.float32
    H = W = 256

    img = jnp.stack((raw1, raw2))
    xpad = jnp.pad(img, ((0, 0), (1, 1), (1, 1)))
    p1m = jnp.stack(
        [xpad[i, dh:dh + H, dw:dw + W]
         for i in range(2) for dh in range(3) for dw in range(3)],
        axis=-1).reshape(_HW, 18)
    w1m = conv1_w.reshape(32, 18).T
    h1 = _mm_call(p1m, w1m, conv1_b, act=True, blk=4096)

    fpad = jnp.pad(h1.reshape(H, W, 32),
                   ((1, 1), (8, 8), (0, 0))).reshape(258 * _WP, 32)
    fpp = jnp.pad(fpad, ((1, 1), (0, 0)))
    nq = 258 * _WP
    g = jnp.concatenate([fpp[0:nq], fpp[1:nq + 1], fpp[2:nq + 2]], axis=1)
    gp = jnp.pad(g, ((_WP, _GPR - nq - _WP), (0, 0)))
    w2m = conv2_w.transpose(2, 3, 1, 0).reshape(288, _C)
    pix = _conv2_call(gp, w2m, conv2_b)

    sums, cnts = _sc_pool(pix, sp_indices.astype(jnp.int32))
    node_features = _merge_call(sums[0], sums[1], cnts[0], cnts[1])

    src = edge_index[0].astype(jnp.int32)
    dst = edge_index[1].astype(jnp.int32)
    ew2 = jnp.concatenate((edge_weights, edge_weights), axis=0)

    agg1 = _sc_scatter(node_features, src, dst, angles)
    nf1, g1 = _node_call(agg1[0], agg1[1], nc1_w, nc1_b, ec1_w)

    agg2 = _sc_scatter(nf1, src, dst, angles)
    s1 = _sc_pair(g1, src, dst)
    w_top = ec2_w[:_C]
    w_bot = ec2_w[_C:]
    _, g2 = _node_call(agg2[0], agg2[1], nc2_w, nc2_b, w_top)
    s2 = _sc_pair(g2, src, dst)

    pv = p1_w @ p2_w
    qv = q1_w[:_C + _F1D] @ q2_w
    kq = (q1_w[_C + _F1D] @ q2_w)[0]
    pbias = p1_b @ p2_w + p2_b
    qbias = q1_b @ q2_w + q2_b
    vmat = jnp.concatenate([pv[:_C], qv[:_C]], axis=1)
    v1d = jnp.concatenate([pv[_C:], qv[_C:]], axis=1)
    bvec = jnp.stack([pbias[0], qbias[0]]).reshape(1, 2)

    rkey = jax.random.key(42)
    us = [jax.random.uniform(jax.random.fold_in(rkey, i), (_E,),
                             jnp.float32).reshape(_E, 1)
          for i in (0, 1, 1000)]
    outs = _final_call(s1, s2, ew2.reshape(_E, 1), edge_features_1d,
                       w_bot, ec1_b.reshape(1, _C), ec2_b.reshape(1, _C),
                       vmat, v1d, bvec, us[0], us[1], us[2],
                       action_behav.reshape(_E, 1), kq.reshape(1, 1))
    p, q, v, sampled_action, q_prime = [o.reshape(_E) for o in outs]
    return (p, q, v, sampled_action, q_prime)


# speedup vs baseline: 1.7092x; 1.7092x over previous
import functools

import jax
import jax.numpy as jnp
from jax import lax
from jax.experimental import pallas as pl
from jax.experimental.pallas import tpu as pltpu
from jax.experimental.pallas import tpu_sc as plsc

_N = 10000
_NP = 10112
_HW = 65536
_E = 160000
_C = 128
_CA = 144
_F1D = 16
_NC = 2
_NS = 16
_NW = _NC * _NS
_CH = 128


def _leaky(x):
    return jnp.where(x >= 0, x, 0.01 * x)


def _mm_call(x, w, b, act, blk):
    m, k = x.shape
    n = w.shape[1]
    assert m % blk == 0

    def body(x_ref, w_ref, b_ref, o_ref):
        y = jnp.dot(x_ref[...], w_ref[...], preferred_element_type=jnp.float32)
        y = y + b_ref[...]
        if act:
            y = _leaky(y)
        o_ref[...] = y

    return pl.pallas_call(
        body,
        grid=(m // blk,),
        in_specs=[
            pl.BlockSpec((blk, k), lambda i: (i, 0)),
            pl.BlockSpec((k, n), lambda i: (0, 0)),
            pl.BlockSpec((1, n), lambda i: (0, 0)),
        ],
        out_specs=pl.BlockSpec((blk, n), lambda i: (i, 0)),
        out_shape=jax.ShapeDtypeStruct((m, n), jnp.float32),
    )(x, w, b.reshape(1, n))


_WP = 272
_CB = 5888
_CNB = 12
_GPR = 272 + _CB * _CNB + 272


def _conv2_call(gp, w2m, b2):
    def body(gp_ref, w_ref, b_ref, o_ref):
        i = pl.program_id(0)
        acc = jnp.zeros((_CB, _C), jnp.float32) + b_ref[...]
        for dh in range(3):
            blk = gp_ref[pl.ds(i * _CB + _WP * dh, _CB), :]
            acc = acc + jnp.dot(blk, w_ref[pl.ds(96 * dh, 96), :],
                                preferred_element_type=jnp.float32)
        o_ref[...] = acc

    return pl.pallas_call(
        body,
        grid=(_CNB,),
        in_specs=[
            pl.BlockSpec((_GPR, 96), lambda i: (0, 0)),
            pl.BlockSpec((288, _C), lambda i: (0, 0)),
            pl.BlockSpec((1, _C), lambda i: (0, 0)),
        ],
        out_specs=pl.BlockSpec((_CB, _C), lambda i: (i, 0)),
        out_shape=jax.ShapeDtypeStruct((_CB * _CNB, _C), jnp.float32),
    )(gp, w2m, b2.reshape(1, _C))


def _merge_call(s0, s1, c0, c1):
    def body(a_ref, b_ref, ca_ref, cb_ref, o_ref):
        t = a_ref[...] + b_ref[...]
        cnt = jnp.maximum(ca_ref[...] + cb_ref[...], 1.0)
        o_ref[...] = t / cnt

    return pl.pallas_call(
        body,
        out_shape=jax.ShapeDtypeStruct((_NP, _C), jnp.float32),
    )(s0, s1, c0, c1)


def _node_call(a0, a1, w1, b1, w2):
    blk = 1264

    def body(a0_ref, a1_ref, w1_ref, b1_ref, w2_ref, nf_ref, g_ref):
        x = a0_ref[...] + a1_ref[...]
        nf = _leaky(jnp.dot(x, w1_ref[...], preferred_element_type=jnp.float32)
                    + b1_ref[...])
        nf_ref[...] = nf
        g_ref[...] = jnp.dot(nf, w2_ref[...], preferred_element_type=jnp.float32)

    return pl.pallas_call(
        body,
        grid=(_NP // blk,),
        in_specs=[
            pl.BlockSpec((blk, _C), lambda i: (i, 0)),
            pl.BlockSpec((blk, _C), lambda i: (i, 0)),
            pl.BlockSpec((_C, _C), lambda i: (0, 0)),
            pl.BlockSpec((1, _C), lambda i: (0, 0)),
            pl.BlockSpec((_C, _C), lambda i: (0, 0)),
        ],
        out_specs=[
            pl.BlockSpec((blk, _C), lambda i: (i, 0)),
            pl.BlockSpec((blk, _C), lambda i: (i, 0)),
        ],
        out_shape=[
            jax.ShapeDtypeStruct((_NP, _C), jnp.float32),
            jax.ShapeDtypeStruct((_NP, _C), jnp.float32),
        ],
    )(a0, a1, w1, b1.reshape(1, _C), w2)


def _final_call(s1, s2, ew2, ef1d, w_bot, ec1_b, ec2_b, vmat, v1d, bvec):
    blk = 3200

    def body(s1_ref, s2_ref, ew_ref, e1d_ref, wb_ref, b1_ref, b2_ref,
             v_ref, v1_ref, bv_ref, o_ref):
        ew = ew_ref[...]
        ef1 = _leaky(ew * s1_ref[...] + b1_ref[...])
        z = _leaky(ew * s2_ref[...] +
                   jnp.dot(ef1.astype(jnp.bfloat16),
                           wb_ref[...].astype(jnp.bfloat16),
                           preferred_element_type=jnp.float32)
                   + b2_ref[...])
        o = (jnp.dot(z, v_ref[...], preferred_element_type=jnp.float32)
             + jnp.dot(e1d_ref[...], v1_ref[...],
                       preferred_element_type=jnp.float32)
             + bv_ref[...])
        o_ref[...] = o

    return pl.pallas_call(
        body,
        grid=(_E // blk,),
        in_specs=[
            pl.BlockSpec((blk, _C), lambda i: (i, 0)),
            pl.BlockSpec((blk, _C), lambda i: (i, 0)),
            pl.BlockSpec((blk, 1), lambda i: (i, 0)),
            pl.BlockSpec((blk, _F1D), lambda i: (i, 0)),
            pl.BlockSpec((_C, _C), lambda i: (0, 0)),
            pl.BlockSpec((1, _C), lambda i: (0, 0)),
            pl.BlockSpec((1, _C), lambda i: (0, 0)),
            pl.BlockSpec((_C, 2), lambda i: (0, 0)),
            pl.BlockSpec((_F1D, 2), lambda i: (0, 0)),
            pl.BlockSpec((1, 2), lambda i: (0, 0)),
        ],
        out_specs=pl.BlockSpec((blk, 2), lambda i: (i, 0)),
        out_shape=jax.ShapeDtypeStruct((_E, 2), jnp.float32),
    )(s1, s2, ew2, ef1d, w_bot, ec1_b, ec2_b, vmat, v1d, bvec)


_MESH = dict(core_axis_name="c", subcore_axis_name="s")


def _zero_vmem(ref, nrows, ncols):
    zv = jnp.zeros((16,), jnp.float32)
    npc = ncols // 16

    def zb(i, carry):
        ref[i // npc, pl.ds((i % npc) * 16, 16)] = zv
        return carry

    lax.fori_loop(0, nrows * npc, zb, 0)


def _zero_acc(acc, zbuf, sid, rows_per_tile):
    base = sid * rows_per_tile
    nfull = rows_per_tile // _CH
    rem = rows_per_tile - nfull * _CH
    for t in range(nfull):
        pltpu.sync_copy(zbuf, acc.at[pl.ds(base + t * _CH, _CH)])
    if rem:
        pltpu.sync_copy(zbuf.at[pl.ds(0, rem)],
                        acc.at[pl.ds(base + nfull * _CH, rem)])


def _sc_pool(pix, sp_indices):
    mesh = plsc.VectorSubcoreMesh(**_MESH)
    nchunk = _HW // _CH // _NW
    rpt = _NP // _NS

    @functools.partial(
        pl.kernel,
        out_type=[
            jax.ShapeDtypeStruct((_NC, _NP, _C), jnp.float32),
            jax.ShapeDtypeStruct((_NC, _NP, _C), jnp.float32),
        ],
        mesh=mesh,
        scratch_types=[
            pltpu.VMEM((_CH, _C), jnp.float32),
            pltpu.VMEM((_CH, _C), jnp.float32),
            pltpu.VMEM((_CH,), jnp.int32),
            pltpu.VMEM_SHARED((_NP, _C), jnp.float32),
            pltpu.SemaphoreType.DMA,
        ],
    )
    def k(pix_hbm, sp_hbm, sum_hbm, cnt_hbm, zbuf, rows, idx, acc, sem):
        c = lax.axis_index("c")
        s = lax.axis_index("s")
        w = c * _NS + s
        _zero_vmem(zbuf, _CH, _C)
        _zero_acc(acc, zbuf, s, rpt)
        plsc.subcore_barrier()

        def chunk(j, carry):
            k = w * nchunk + j
            off = pl.multiple_of(k * _CH, 8)
            prow = pl.multiple_of(_WP * (k // 2) + _WP + _CH * (k % 2) + 8, 8)
            pltpu.sync_copy(sp_hbm.at[pl.ds(off, _CH)], idx)
            pltpu.sync_copy(pix_hbm.at[pl.ds(prow, _CH)], rows)
            pltpu.sync_copy(rows, acc.at[idx], add=True)
            return carry

        lax.fori_loop(0, nchunk, chunk, 0)
        plsc.subcore_barrier()
        pltpu.sync_copy(acc.at[pl.ds(s * rpt, rpt)],
                        sum_hbm.at[c, pl.ds(s * rpt, rpt)])
        plsc.subcore_barrier()
        _zero_acc(acc, zbuf, s, rpt)
        one = jnp.ones((16,), jnp.float32)

        def ob(i, carry):
            rows[i // 8, pl.ds((i % 8) * 16, 16)] = one
            return carry

        lax.fori_loop(0, _CH * 8, ob, 0)
        plsc.subcore_barrier()

        def chunk2(j, carry):
            off = pl.multiple_of((w * nchunk + j) * _CH, 8)
            pltpu.sync_copy(sp_hbm.at[pl.ds(off, _CH)], idx)
            pltpu.sync_copy(rows, acc.at[idx], add=True)
            return carry

        lax.fori_loop(0, nchunk, chunk2, 0)
        plsc.subcore_barrier()
        pltpu.sync_copy(acc.at[pl.ds(s * rpt, rpt)],
                        cnt_hbm.at[c, pl.ds(s * rpt, rpt)])

    return k(pix, sp_indices)


def _edge_off(w, j):
    return pl.multiple_of((w + _NW * j) * _CH, 8)


def _edge_nch(w):
    return jnp.where(w < 2, 40, 39)


def _edge_pipeline(w, prefetch_fn, process_fn):
    nch = _edge_nch(w)
    prefetch_fn(_edge_off(w, 0), 0)

    def body(jj, carry):
        for b in range(2):
            j = jj * 2 + b

            @pl.when(j < nch)
            def _():
                @pl.when(j + 1 < nch)
                def _():
                    prefetch_fn(_edge_off(w, j + 1), 1 - b)

                process_fn(_edge_off(w, j), b)
        return carry

    lax.fori_loop(0, 20, body, 0)


def _sc_scatter(table, src, dst, ang):
    mesh = plsc.VectorSubcoreMesh(**_MESH)
    rpt = _NP // _NS

    @functools.partial(
        pl.kernel,
        out_type=jax.ShapeDtypeStruct((_NC, _NP, _C), jnp.float32),
        mesh=mesh,
        scratch_types=[
            pltpu.VMEM((_CH, _C), jnp.float32),
            pltpu.VMEM((_CH, _C), jnp.float32),
            pltpu.VMEM((_CH, _C), jnp.float32),
            pltpu.VMEM((_CH,), jnp.int32),
            pltpu.VMEM((_CH,), jnp.int32),
            pltpu.VMEM((_CH,), jnp.int32),
            pltpu.VMEM((_CH,), jnp.int32),
            pltpu.VMEM((_CH,), jnp.float32),
            pltpu.VMEM((_CH,), jnp.float32),
            pltpu.VMEM_SHARED((_NP, _C), jnp.float32),
            pltpu.SemaphoreType.DMA,
            pltpu.SemaphoreType.DMA,
        ],
    )
    def k(tab_hbm, src_hbm, dst_hbm, ang_hbm, out_hbm,
          zbuf, rows0, rows1, si0, si1, di0, di1, av0, av1, acc,
          semg0, semg1):
        c = lax.axis_index("c")
        s = lax.axis_index("s")
        w = c * _NS + s
        rows = (rows0, rows1)
        si = (si0, si1)
        di = (di0, di1)
        av = (av0, av1)
        semg = (semg0, semg1)
        _zero_vmem(zbuf, _CH, _C)
        _zero_acc(acc, zbuf, s, rpt)
        plsc.subcore_barrier()

        dnums = lax.GatherDimensionNumbers(
            offset_dims=(), collapsed_slice_dims=(0,), start_index_map=(0,))

        def prefetch(off, b):
            pltpu.sync_copy(src_hbm.at[pl.ds(off, _CH)], si[b])
            pltpu.sync_copy(dst_hbm.at[pl.ds(off, _CH)], di[b])
            pltpu.sync_copy(ang_hbm.at[pl.ds(off, _CH)], av[b])
            pltpu.async_copy(tab_hbm.at[si[b]], rows[b], semg[b])

        def process(off, b):
            pltpu.make_async_copy(tab_hbm.at[si[b]], rows[b], semg[b]).wait()

            def rb(g, carry):
                a16 = av[b][pl.ds(g * 16, 16)]
                for l in range(16):
                    a = lax.gather(
                        a16, jnp.full((16, 1), l, jnp.int32), dnums, (1,),
                        mode=lax.GatherScatterMode.PROMISE_IN_BOUNDS)
                    r = g * 16 + l
                    for k8 in range(_C // 16):
                        sl = pl.ds(k8 * 16, 16)
                        rows[b][r, sl] = rows[b][r, sl] * a
                return carry

            lax.fori_loop(0, _CH // 16, rb, 0)
            pltpu.sync_copy(rows[b], acc.at[di[b]], add=True)

        _edge_pipeline(w, prefetch, process)
        plsc.subcore_barrier()
        pltpu.sync_copy(acc.at[pl.ds(s * rpt, rpt)],
                        out_hbm.at[c, pl.ds(s * rpt, rpt)])

    return k(table, src, dst, ang)


def _sc_pair(table, src, dst):
    mesh = plsc.VectorSubcoreMesh(**_MESH)

    @functools.partial(
        pl.kernel,
        out_type=jax.ShapeDtypeStruct((_E, _C), jnp.float32),
        mesh=mesh,
        scratch_types=[
            pltpu.VMEM((_CH, _C), jnp.float32),
            pltpu.VMEM((_CH, _C), jnp.float32),
            pltpu.VMEM((_CH, _C), jnp.float32),
            pltpu.VMEM((_CH, _C), jnp.float32),
            pltpu.VMEM((_CH,), jnp.int32),
            pltpu.VMEM((_CH,), jnp.int32),
            pltpu.VMEM((_CH,), jnp.int32),
            pltpu.VMEM((_CH,), jnp.int32),
            pltpu.SemaphoreType.DMA,
            pltpu.SemaphoreType.DMA,
        ],
    )
    def k(tab_hbm, src_hbm, dst_hbm, out_hbm,
          ra0, ra1, rb0, rb1, si0, si1, di0, di1, semg0, semg1):
        c = lax.axis_index("c")
        s = lax.axis_index("s")
        w = c * _NS + s
        ra = (ra0, ra1)
        rb = (rb0, rb1)
        si = (si0, si1)
        di = (di0, di1)
        semg = (semg0, semg1)

        def prefetch(off, b):
            pltpu.sync_copy(src_hbm.at[pl.ds(off, _CH)], si[b])
            pltpu.sync_copy(dst_hbm.at[pl.ds(off, _CH)], di[b])
            pltpu.async_copy(tab_hbm.at[si[b]], ra[b], semg[b])
            pltpu.async_copy(tab_hbm.at[di[b]], rb[b], semg[b])

        def process(off, b):
            pltpu.make_async_copy(tab_hbm.at[si[b]], ra[b], semg[b]).wait()
            pltpu.make_async_copy(tab_hbm.at[di[b]], rb[b], semg[b]).wait()

            def rr(r, carry):
                for k8 in range(_C // 16):
                    sl = pl.ds(k8 * 16, 16)
                    ra[b][r, sl] = ra[b][r, sl] + rb[b][r, sl]
                return carry

            lax.fori_loop(0, _CH, rr, 0)
            pltpu.sync_copy(ra[b], out_hbm.at[pl.ds(off, _CH)])

        _edge_pipeline(w, prefetch, process)

    return k(table, src, dst)


def kernel(edge_weights, raw1, raw2, action_behav, angles, edge_features_1d,
           conv1_w, conv1_b, conv2_w, conv2_b, nc1_w, nc1_b, ec1_w, ec1_b,
           nc2_w, nc2_b, ec2_w, ec2_b, p1_w, p1_b, p2_w, p2_b, q1_w, q1_b,
           q2_w, q2_b, sp_indices, edge_index):
    f32 = jnp.float32
    H = W = 256

    img = jnp.stack((raw1, raw2))
    xpad = jnp.pad(img, ((0, 0), (1, 1), (1, 1)))
    p1m = jnp.stack(
        [xpad[i, dh:dh + H, dw:dw + W]
         for i in range(2) for dh in range(3) for dw in range(3)],
        axis=-1).reshape(_HW, 18)
    w1m = conv1_w.reshape(32, 18).T
    h1 = _mm_call(p1m, w1m, conv1_b, act=True, blk=4096)

    fpad = jnp.pad(h1.reshape(H, W, 32),
                   ((1, 1), (8, 8), (0, 0))).reshape(258 * _WP, 32)
    fpp = jnp.pad(fpad, ((1, 1), (0, 0)))
    nq = 258 * _WP
    g = jnp.concatenate([fpp[0:nq], fpp[1:nq + 1], fpp[2:nq + 2]], axis=1)
    gp = jnp.pad(g, ((_WP, _GPR - nq - _WP), (0, 0)))
    w2m = conv2_w.transpose(2, 3, 1, 0).reshape(288, _C)
    pix = _conv2_call(gp, w2m, conv2_b)

    sums, cnts = _sc_pool(pix, sp_indices.astype(jnp.int32))
    node_features = _merge_call(sums[0], sums[1], cnts[0], cnts[1])

    src = edge_index[0].astype(jnp.int32)
    dst = edge_index[1].astype(jnp.int32)
    ew2 = jnp.concatenate((edge_weights, edge_weights), axis=0)

    agg1 = _sc_scatter(node_features, src, dst, angles)
    nf1, g1 = _node_call(agg1[0], agg1[1], nc1_w, nc1_b, ec1_w)

    agg2 = _sc_scatter(nf1, src, dst, angles)
    s1 = _sc_pair(g1, src, dst)
    w_top = ec2_w[:_C]
    w_bot = ec2_w[_C:]
    _, g2 = _node_call(agg2[0], agg2[1], nc2_w, nc2_b, w_top)
    s2 = _sc_pair(g2, src, dst)

    pv = p1_w @ p2_w
    qv = q1_w[:_C + _F1D] @ q2_w
    kq = (q1_w[_C + _F1D] @ q2_w)[0]
    pbias = p1_b @ p2_w + p2_b
    qbias = q1_b @ q2_w + q2_b
    vmat = jnp.concatenate([pv[:_C], qv[:_C]], axis=1)
    v1d = jnp.concatenate([pv[_C:], qv[_C:]], axis=1)
    bvec = jnp.stack([pbias[0], qbias[0]]).reshape(1, 2)

    out = _final_call(s1, s2, ew2.reshape(_E, 1), edge_features_1d,
                      w_bot, ec1_b.reshape(1, _C), ec2_b.reshape(1, _C),
                      vmat, v1d, bvec)
    p = jax.nn.sigmoid(out[:, 0])
    base = out[:, 1]

    sigma = 0.2
    rkey = jax.random.key(42)

    def tsample(i):
        a = (0.0 - p) / sigma
        b = (1.0 - p) / sigma
        t = jax.random.truncated_normal(jax.random.fold_in(rkey, i), a, b,
                                        p.shape, p.dtype)
        return p + sigma * t

    act0 = tsample(0)
    act1 = tsample(1)
    sampled_action = tsample(1000)
    mean_act = 0.5 * (act0 + act1)
    v = base + kq * action_behav
    q = base + kq * (2.0 * action_behav - mean_act)
    q_prime = base + kq * (2.0 * sampled_action - mean_act)
    return (p, q, v, sampled_action, q_prime)
